# Initial kernel scaffold; baseline (speedup 1.0000x reference)
import jax
import jax.numpy as jnp
from jax.experimental import pallas as pl


def kernel(x, edge_index, W, b):
    raise NotImplementedError("write your pallas kernel here")


# speedup vs baseline: 4.3240x; 4.3240x over previous
import functools

import jax
import jax.numpy as jnp
from jax import lax
from jax.experimental import pallas as pl
from jax.experimental.pallas import tpu as pltpu
from jax.experimental.pallas import tpu_sc as plsc

N = 10000
E = 320000
D = 128

_NC = 2
_NS = 16
_NW = _NC * _NS
_EPT = E // _NW
_C = 80
_NCHUNK = _EPT // _C
_U = 80
_NU = N // _U
_KMAX = -(-_NU // _NS)
_L = 16


def _arccosh(x):
    return jnp.log(x + jnp.sqrt((x - 1.0) * (x + 1.0)))


def _tangent_of(y_s):
    nsq = jnp.sum(y_s * y_s, axis=-1, keepdims=True)
    x0 = jnp.maximum(jnp.sqrt(1.0 + nsq), 1.0 + 1e-7)
    r = _arccosh(x0)
    n = jnp.sqrt(nsq)
    return r * y_s / jnp.maximum(n, 1e-7)


def _exp0_spatial(v):
    rsq = jnp.sum(v * v, axis=-1, keepdims=True)
    r = jnp.sqrt(rsq)
    safe_r = jnp.maximum(r, 1e-7)
    e = jnp.exp(r)
    sinh = 0.5 * (e - 1.0 / e)
    coef = jnp.where(r > 1e-7, sinh / safe_r, 1.0)
    return coef * v, r


def _transform_body(xs_ref, w_ref, b_ref, vt_ref):
    xs = xs_ref[...]
    u = _tangent_of(xs)
    y = lax.dot_general(u, w_ref[...], (((1,), (1,)), ((), ())),
                        preferred_element_type=jnp.float32) + b_ref[...]
    ys, r = _exp0_spatial(y)
    e = jnp.exp(r)
    time = jnp.maximum(0.5 * (e + 1.0 / e), 1.0 + 1e-7)
    rt = _arccosh(time)
    nt = jnp.sqrt(jnp.sum(ys * ys, axis=-1, keepdims=True))
    vt_ref[...] = rt * ys / jnp.maximum(nt, 1e-7)


def _tail_body(s0_ref, s1_ref, vt_ref, c_ref, xs_out_ref, x0_out_ref):
    vt = vt_ref[...]
    s = s0_ref[...] + s1_ref[...] + vt
    cnt = jnp.sum(c_ref[...], axis=0) + 1.0
    mean = s / jnp.maximum(cnt, 1.0)
    xa, _ = _exp0_spatial(mean)
    v = jnp.maximum(_tangent_of(xa), 0.0)
    xo, _ = _exp0_spatial(v)
    xs_out_ref[...] = xo
    x0_out_ref[...] = jnp.sqrt(1.0 + jnp.sum(xo * xo, axis=-1, keepdims=True))


def _sc_aggregate(vt, src, dst, zrow, zflat):
    mesh = plsc.VectorSubcoreMesh(core_axis_name="c", subcore_axis_name="s")

    @functools.partial(
        pl.kernel,
        out_type=(jax.ShapeDtypeStruct((_NC * N, D), jnp.float32),
                  jax.ShapeDtypeStruct((_NW * N,), jnp.float32)),
        mesh=mesh,
        compiler_params=pltpu.CompilerParams(needs_layout_passes=False),
        scratch_types=[
            pltpu.VMEM((_C,), jnp.int32),
            pltpu.VMEM((_C,), jnp.int32),
            pltpu.VMEM((_C,), jnp.int32),
            pltpu.VMEM((_C, D), jnp.float32),
            pltpu.VMEM((N,), jnp.float32),
            pltpu.VMEM_SHARED((N, D), jnp.float32),
            pltpu.SemaphoreType.DMA,
        ],
    )
    def agg(vt_hbm, src_hbm, dst_hbm, zrow_hbm, zflat_hbm,
            sums_hbm, cnts_hbm,
            src_v, dst_v, dstv2_v, rows_v, cnt_v, acc_sh, sem):
        cid = lax.axis_index("c")
        sid = lax.axis_index("s")
        wid = cid * _NS + sid

        pltpu.sync_copy(zflat_hbm, cnt_v)

        pltpu.sync_copy(zrow_hbm, rows_v)
        for k in range(_KMAX):
            u = sid + _NS * k

            @pl.when(u < _NU)
            def _zero_unit():
                pltpu.sync_copy(rows_v, acc_sh.at[pl.ds(u * _U, _U)])

        plsc.subcore_barrier()

        one16 = jnp.ones((_L,), jnp.float32)

        def chunk(i, carry):
            base = wid * _EPT + i * _C
            pltpu.sync_copy(src_hbm.at[pl.ds(base, _C)], src_v)
            pltpu.sync_copy(dst_hbm.at[pl.ds(base, _C)], dst_v)
            pltpu.sync_copy(dst_hbm.at[pl.ds(base, _C)], dstv2_v)
            pltpu.async_copy(vt_hbm.at[src_v], rows_v, sem).wait()
            pltpu.sync_copy(rows_v, acc_sh.at[dst_v], add=True)
            for j in range(_C // _L):
                dvals = dstv2_v[pl.ds(j * _L, _L)]
                plsc.addupdate_scatter(cnt_v, [dvals], one16)
            return carry

        lax.fori_loop(0, _NCHUNK, chunk, 0)
        plsc.subcore_barrier()

        for k in range(_KMAX):
            u = sid + _NS * k

            @pl.when(u < _NU)
            def _write_unit():
                pltpu.sync_copy(acc_sh.at[pl.ds(u * _U, _U)], rows_v)
                pltpu.sync_copy(rows_v, sums_hbm.at[pl.ds(cid * N + u * _U, _U)])

        pltpu.sync_copy(cnt_v, cnts_hbm.at[pl.ds(wid * N, N)])

    return agg(vt, src, dst, zrow, zflat)


def kernel(x, edge_index, W, b):
    xs = x[:, 1:]
    b2 = b.reshape(1, D)
    bn = 1000
    grid = N // bn

    vt = pl.pallas_call(
        _transform_body,
        grid=(grid,),
        in_specs=[
            pl.BlockSpec((bn, D), lambda i: (i, 0)),
            pl.BlockSpec((D, D), lambda i: (0, 0)),
            pl.BlockSpec((1, D), lambda i: (0, 0)),
        ],
        out_specs=pl.BlockSpec((bn, D), lambda i: (i, 0)),
        out_shape=jax.ShapeDtypeStruct((N, D), jnp.float32),
    )(xs, W, b2)

    zrow = jnp.zeros((_C, D), jnp.float32)
    zflat = jnp.zeros((N,), jnp.float32)
    sums, cnts = _sc_aggregate(vt, edge_index[0], edge_index[1], zrow, zflat)
    cnts = cnts.reshape(_NW, N, 1)

    xs_o, x0_o = pl.pallas_call(
        _tail_body,
        grid=(grid,),
        in_specs=[
            pl.BlockSpec((bn, D), lambda i: (i, 0)),
            pl.BlockSpec((bn, D), lambda i: (i, 0)),
            pl.BlockSpec((bn, D), lambda i: (i, 0)),
            pl.BlockSpec((_NW, bn, 1), lambda i: (0, i, 0)),
        ],
        out_specs=[
            pl.BlockSpec((bn, D), lambda i: (i, 0)),
            pl.BlockSpec((bn, 1), lambda i: (i, 0)),
        ],
        out_shape=[
            jax.ShapeDtypeStruct((N, D), jnp.float32),
            jax.ShapeDtypeStruct((N, 1), jnp.float32),
        ],
    )(sums[:N], sums[N:], vt, cnts)

    return jnp.concatenate([x0_o, xs_o], axis=-1)


# speedup vs baseline: 6.3602x; 1.4709x over previous
import functools

import jax
import jax.numpy as jnp
from jax import lax
from jax.experimental import pallas as pl
from jax.experimental.pallas import tpu as pltpu
from jax.experimental.pallas import tpu_sc as plsc

N = 10000
E = 320000
D = 128

_NC = 2
_NS = 16
_NW = _NC * _NS
_EPT = E // _NW
_C = 40
_NBUF = 5
_NGRP = _EPT // (_C * _NBUF)
_U = 40
_NU = N // _U
_KMAX = -(-_NU // _NS)
_L = 16


def _arccosh(x):
    return jnp.log(x + jnp.sqrt((x - 1.0) * (x + 1.0)))


def _tangent_of(y_s):
    nsq = jnp.sum(y_s * y_s, axis=-1, keepdims=True)
    x0 = jnp.maximum(jnp.sqrt(1.0 + nsq), 1.0 + 1e-7)
    r = _arccosh(x0)
    n = jnp.sqrt(nsq)
    return r * y_s / jnp.maximum(n, 1e-7)


def _exp0_spatial(v):
    rsq = jnp.sum(v * v, axis=-1, keepdims=True)
    r = jnp.sqrt(rsq)
    safe_r = jnp.maximum(r, 1e-7)
    e = jnp.exp(r)
    sinh = 0.5 * (e - 1.0 / e)
    coef = jnp.where(r > 1e-7, sinh / safe_r, 1.0)
    return coef * v, r


def _transform_body(xs_ref, w_ref, b_ref, vt_ref):
    xs = xs_ref[...]
    u = _tangent_of(xs)
    y = lax.dot_general(u, w_ref[...], (((1,), (1,)), ((), ())),
                        preferred_element_type=jnp.float32) + b_ref[...]
    ys, r = _exp0_spatial(y)
    e = jnp.exp(r)
    time = jnp.maximum(0.5 * (e + 1.0 / e), 1.0 + 1e-7)
    rt = _arccosh(time)
    nt = jnp.sqrt(jnp.sum(ys * ys, axis=-1, keepdims=True))
    vt_ref[...] = rt * ys / jnp.maximum(nt, 1e-7)


def _tail_body(s0_ref, s1_ref, vt_ref, c_ref, xs_out_ref, x0_out_ref):
    vt = vt_ref[...]
    s = s0_ref[...] + s1_ref[...] + vt
    cnt = jnp.sum(c_ref[...], axis=0) + 1.0
    mean = s / jnp.maximum(cnt, 1.0)
    xa, _ = _exp0_spatial(mean)
    v = jnp.maximum(_tangent_of(xa), 0.0)
    xo, _ = _exp0_spatial(v)
    xs_out_ref[...] = xo
    x0_out_ref[...] = jnp.sqrt(1.0 + jnp.sum(xo * xo, axis=-1, keepdims=True))


def _sc_aggregate(vt, src, dst, zrow, zflat):
    mesh = plsc.VectorSubcoreMesh(core_axis_name="c", subcore_axis_name="s")

    @functools.partial(
        pl.kernel,
        out_type=(jax.ShapeDtypeStruct((_NC * N, D), jnp.float32),
                  jax.ShapeDtypeStruct((_NW * N,), jnp.float32)),
        mesh=mesh,
        compiler_params=pltpu.CompilerParams(needs_layout_passes=False),
        scratch_types=[
            [pltpu.VMEM((_C,), jnp.int32)] * _NBUF,
            [pltpu.VMEM((_C,), jnp.int32)] * _NBUF,
            [pltpu.VMEM((_C, D), jnp.float32)] * _NBUF,
            pltpu.VMEM((_EPT,), jnp.int32),
            pltpu.VMEM((N,), jnp.float32),
            pltpu.VMEM_SHARED((N, D), jnp.float32),
            pltpu.SemaphoreType.DMA,
            pltpu.SemaphoreType.DMA,
            pltpu.SemaphoreType.DMA,
        ],
    )
    def agg(vt_hbm, src_hbm, dst_hbm, zrow_hbm, zflat_hbm,
            sums_hbm, cnts_hbm,
            src_v, dst_v, rows_v, dall_v, cnt_v, acc_sh,
            sem_i, sem_g, sem_s):
        cid = lax.axis_index("c")
        sid = lax.axis_index("s")
        wid = cid * _NS + sid

        pltpu.sync_copy(zflat_hbm, cnt_v)
        pltpu.sync_copy(dst_hbm.at[pl.ds(wid * _EPT, _EPT)], dall_v)

        pltpu.sync_copy(zrow_hbm, rows_v[0])
        for k in range(_KMAX):
            u = sid + _NS * k

            @pl.when(u < _NU)
            def _zero_unit():
                pltpu.sync_copy(rows_v[0], acc_sh.at[pl.ds(u * _U, _U)])

        plsc.subcore_barrier()

        def group(g, carry):
            base0 = wid * _EPT + g * (_C * _NBUF)
            hi, hg, hs = [], [], []
            for b in range(_NBUF):
                base = base0 + b * _C
                hi.append(pltpu.async_copy(
                    src_hbm.at[pl.ds(base, _C)], src_v[b], sem_i))
                hi.append(pltpu.async_copy(
                    dst_hbm.at[pl.ds(base, _C)], dst_v[b], sem_i))
            for b in range(_NBUF):
                hi[2 * b].wait()
                hg.append(pltpu.async_copy(
                    vt_hbm.at[src_v[b]], rows_v[b], sem_g))
            for b in range(_NBUF):
                hg[b].wait()
                hi[2 * b + 1].wait()
                hs.append(pltpu.async_copy(
                    rows_v[b], acc_sh.at[dst_v[b]], sem_s, add=True))
            for b in range(_NBUF):
                hs[b].wait()
            return carry

        lax.fori_loop(0, _NGRP, group, 0)

        one16 = jnp.ones((_L,), jnp.float32)

        def cloop(k, carry):
            dvals = dall_v[pl.ds(k * _L, _L)]
            plsc.addupdate_scatter(cnt_v, [dvals], one16)
            return carry

        lax.fori_loop(0, _EPT // _L, cloop, 0)
        plsc.subcore_barrier()

        for k in range(_KMAX):
            u = sid + _NS * k

            @pl.when(u < _NU)
            def _write_unit():
                pltpu.sync_copy(acc_sh.at[pl.ds(u * _U, _U)], rows_v[0])
                pltpu.sync_copy(rows_v[0], sums_hbm.at[pl.ds(cid * N + u * _U, _U)])

        pltpu.sync_copy(cnt_v, cnts_hbm.at[pl.ds(wid * N, N)])

    return agg(vt, src, dst, zrow, zflat)


def kernel(x, edge_index, W, b):
    xs = x[:, 1:]
    b2 = b.reshape(1, D)
    bn = 1000
    grid = N // bn

    vt = pl.pallas_call(
        _transform_body,
        grid=(grid,),
        in_specs=[
            pl.BlockSpec((bn, D), lambda i: (i, 0)),
            pl.BlockSpec((D, D), lambda i: (0, 0)),
            pl.BlockSpec((1, D), lambda i: (0, 0)),
        ],
        out_specs=pl.BlockSpec((bn, D), lambda i: (i, 0)),
        out_shape=jax.ShapeDtypeStruct((N, D), jnp.float32),
    )(xs, W, b2)

    zrow = jnp.zeros((_U, D), jnp.float32)
    zflat = jnp.zeros((N,), jnp.float32)
    sums, cnts = _sc_aggregate(vt, edge_index[0], edge_index[1], zrow, zflat)
    cnts = cnts.reshape(_NW, N, 1)

    xs_o, x0_o = pl.pallas_call(
        _tail_body,
        grid=(grid,),
        in_specs=[
            pl.BlockSpec((bn, D), lambda i: (i, 0)),
            pl.BlockSpec((bn, D), lambda i: (i, 0)),
            pl.BlockSpec((bn, D), lambda i: (i, 0)),
            pl.BlockSpec((_NW, bn, 1), lambda i: (0, i, 0)),
        ],
        out_specs=[
            pl.BlockSpec((bn, D), lambda i: (i, 0)),
            pl.BlockSpec((bn, 1), lambda i: (i, 0)),
        ],
        out_shape=[
            jax.ShapeDtypeStruct((N, D), jnp.float32),
            jax.ShapeDtypeStruct((N, 1), jnp.float32),
        ],
    )(sums[:N], sums[N:], vt, cnts)

    return jnp.concatenate([x0_o, xs_o], axis=-1)


# speedup vs baseline: 12.1574x; 1.9115x over previous
import functools

import jax
import jax.numpy as jnp
from jax import lax
from jax.experimental import pallas as pl
from jax.experimental.pallas import tpu as pltpu
from jax.experimental.pallas import tpu_sc as plsc

N = 10000
E = 320000
D = 128

_NC = 2
_NS = 16
_NW = _NC * _NS
_EPT = E // _NW
_C = 40
_NBUF = 5
_NGRP = _EPT // (_C * _NBUF)
_U = 40
_NU = N // _U
_KMAX = -(-_NU // _NS)
_L = 16
_NPAD = 10240


def _arccosh(x):
    return jnp.log(x + jnp.sqrt((x - 1.0) * (x + 1.0)))


def _tangent_of(y_s):
    nsq = jnp.sum(y_s * y_s, axis=-1, keepdims=True)
    x0 = jnp.maximum(jnp.sqrt(1.0 + nsq), 1.0 + 1e-7)
    r = _arccosh(x0)
    n = jnp.sqrt(nsq)
    return r * y_s / jnp.maximum(n, 1e-7)


def _exp0_spatial(v):
    rsq = jnp.sum(v * v, axis=-1, keepdims=True)
    r = jnp.sqrt(rsq)
    safe_r = jnp.maximum(r, 1e-7)
    e = jnp.exp(r)
    sinh = 0.5 * (e - 1.0 / e)
    coef = jnp.where(r > 1e-7, sinh / safe_r, 1.0)
    return coef * v, r


def _transform_body(xs_ref, w_ref, b_ref, vt_ref):
    xs = xs_ref[...]
    u = _tangent_of(xs)
    y = lax.dot_general(u, w_ref[...], (((1,), (1,)), ((), ())),
                        preferred_element_type=jnp.float32) + b_ref[...]
    ys, r = _exp0_spatial(y)
    e = jnp.exp(r)
    time = jnp.maximum(0.5 * (e + 1.0 / e), 1.0 + 1e-7)
    rt = _arccosh(time)
    nt = jnp.sqrt(jnp.sum(ys * ys, axis=-1, keepdims=True))
    vt_ref[...] = rt * ys / jnp.maximum(nt, 1e-7)


def _tail_body(s0_ref, s1_ref, vt_ref, c_ref, xs_out_ref, x0_out_ref):
    vt = vt_ref[...]
    s = s0_ref[...] + s1_ref[...] + vt
    c = jnp.sum(c_ref[...], axis=0)
    cnt = c.reshape(_NPAD)[:N][:, None] + 1.0
    mean = s / jnp.maximum(cnt, 1.0)
    xa, _ = _exp0_spatial(mean)
    v = jnp.maximum(_tangent_of(xa), 0.0)
    xo, _ = _exp0_spatial(v)
    xs_out_ref[...] = xo
    x0_out_ref[...] = jnp.sqrt(1.0 + jnp.sum(xo * xo, axis=-1, keepdims=True))


def _sc_aggregate(vt, src, dst, zrow, zflat):
    mesh = plsc.VectorSubcoreMesh(core_axis_name="c", subcore_axis_name="s")

    @functools.partial(
        pl.kernel,
        out_type=(jax.ShapeDtypeStruct((_NC * N, D), jnp.float32),
                  jax.ShapeDtypeStruct((_NW * _NPAD,), jnp.float32)),
        mesh=mesh,
        compiler_params=pltpu.CompilerParams(needs_layout_passes=False),
        scratch_types=[
            [pltpu.VMEM((_C,), jnp.int32)] * _NBUF,
            [pltpu.VMEM((_C,), jnp.int32)] * _NBUF,
            [pltpu.VMEM((_C, D), jnp.float32)] * _NBUF,
            pltpu.VMEM((_EPT,), jnp.int32),
            pltpu.VMEM((_NPAD,), jnp.float32),
            pltpu.VMEM_SHARED((N, D), jnp.float32),
            pltpu.SemaphoreType.DMA,
            pltpu.SemaphoreType.DMA,
            pltpu.SemaphoreType.DMA,
        ],
    )
    def agg(vt_hbm, src_hbm, dst_hbm, zrow_hbm, zflat_hbm,
            sums_hbm, cnts_hbm,
            src_v, dst_v, rows_v, dall_v, cnt_v, acc_sh,
            sem_i, sem_g, sem_s):
        cid = lax.axis_index("c")
        sid = lax.axis_index("s")
        wid = cid * _NS + sid

        pltpu.sync_copy(zflat_hbm, cnt_v)
        pltpu.sync_copy(dst_hbm.at[pl.ds(wid * _EPT, _EPT)], dall_v)

        pltpu.sync_copy(zrow_hbm, rows_v[0])
        for k in range(_KMAX):
            u = sid + _NS * k

            @pl.when(u < _NU)
            def _zero_unit():
                pltpu.sync_copy(rows_v[0], acc_sh.at[pl.ds(u * _U, _U)])

        plsc.subcore_barrier()

        def group(g, carry):
            base0 = wid * _EPT + g * (_C * _NBUF)
            hi, hg, hs = [], [], []
            for b in range(_NBUF):
                base = base0 + b * _C
                hi.append(pltpu.async_copy(
                    src_hbm.at[pl.ds(base, _C)], src_v[b], sem_i))
                hi.append(pltpu.async_copy(
                    dst_hbm.at[pl.ds(base, _C)], dst_v[b], sem_i))
            for b in range(_NBUF):
                hi[2 * b].wait()
                hg.append(pltpu.async_copy(
                    vt_hbm.at[src_v[b]], rows_v[b], sem_g))
            for b in range(_NBUF):
                hg[b].wait()
                hi[2 * b + 1].wait()
                hs.append(pltpu.async_copy(
                    rows_v[b], acc_sh.at[dst_v[b]], sem_s, add=True))
            for b in range(_NBUF):
                hs[b].wait()
            return carry

        lax.fori_loop(0, _NGRP, group, 0)

        one16 = jnp.ones((_L,), jnp.float32)

        def cloop(k, carry):
            dvals = dall_v[pl.ds(k * _L, _L)]
            plsc.addupdate_scatter(cnt_v, [dvals], one16)
            return carry

        lax.fori_loop(0, _EPT // _L, cloop, 0)
        plsc.subcore_barrier()

        for k in range(_KMAX):
            u = sid + _NS * k

            @pl.when(u < _NU)
            def _write_unit():
                pltpu.sync_copy(acc_sh.at[pl.ds(u * _U, _U)], rows_v[0])
                pltpu.sync_copy(rows_v[0], sums_hbm.at[pl.ds(cid * N + u * _U, _U)])

        pltpu.sync_copy(cnt_v, cnts_hbm.at[pl.ds(wid * _NPAD, _NPAD)])

    return agg(vt, src, dst, zrow, zflat)


def kernel(x, edge_index, W, b):
    xs = x[:, 1:]
    b2 = b.reshape(1, D)
    bn = 1000
    grid = N // bn

    vt = pl.pallas_call(
        _transform_body,
        grid=(grid,),
        in_specs=[
            pl.BlockSpec((bn, D), lambda i: (i, 0)),
            pl.BlockSpec((D, D), lambda i: (0, 0)),
            pl.BlockSpec((1, D), lambda i: (0, 0)),
        ],
        out_specs=pl.BlockSpec((bn, D), lambda i: (i, 0)),
        out_shape=jax.ShapeDtypeStruct((N, D), jnp.float32),
    )(xs, W, b2)

    zrow = jnp.zeros((_U, D), jnp.float32)
    zflat = jnp.zeros((_NPAD,), jnp.float32)
    sums, cnts = _sc_aggregate(vt, edge_index[0], edge_index[1], zrow, zflat)
    cnts = cnts.reshape(_NW, _NPAD // D, D)

    xs_o, x0_o = pl.pallas_call(
        _tail_body,
        out_shape=[
            jax.ShapeDtypeStruct((N, D), jnp.float32),
            jax.ShapeDtypeStruct((N, 1), jnp.float32),
        ],
    )(sums[:N], sums[N:], vt, cnts)

    return jnp.concatenate([x0_o, xs_o], axis=-1)


# speedup vs baseline: 13.0854x; 1.0763x over previous
import functools

import jax
import jax.numpy as jnp
from jax import lax
from jax.experimental import pallas as pl
from jax.experimental.pallas import tpu as pltpu
from jax.experimental.pallas import tpu_sc as plsc

N = 10000
E = 320000
D = 128

_NC = 2
_NS = 16
_NW = _NC * _NS
_EPT = E // _NW
_C = 40
_NBUF = 5
_NGRP = _EPT // (_C * _NBUF)
_U = 40
_NU = N // _U
_KMAX = -(-_NU // _NS)
_L = 16
_NPAD = 10240


def _arccosh(x):
    return jnp.log(x + jnp.sqrt((x - 1.0) * (x + 1.0)))


def _tangent_of(y_s):
    nsq = jnp.sum(y_s * y_s, axis=-1, keepdims=True)
    x0 = jnp.maximum(jnp.sqrt(1.0 + nsq), 1.0 + 1e-7)
    r = _arccosh(x0)
    n = jnp.sqrt(nsq)
    return r * y_s / jnp.maximum(n, 1e-7)


def _exp0_spatial(v):
    rsq = jnp.sum(v * v, axis=-1, keepdims=True)
    r = jnp.sqrt(rsq)
    safe_r = jnp.maximum(r, 1e-7)
    e = jnp.exp(r)
    sinh = 0.5 * (e - 1.0 / e)
    coef = jnp.where(r > 1e-7, sinh / safe_r, 1.0)
    return coef * v, r


def _transform_body(xs_ref, w_ref, b_ref, vt_ref):
    xs = xs_ref[...]
    u = _tangent_of(xs)
    y = lax.dot_general(u, w_ref[...], (((1,), (1,)), ((), ())),
                        preferred_element_type=jnp.float32) + b_ref[...]
    ys, r = _exp0_spatial(y)
    e = jnp.exp(r)
    time = jnp.maximum(0.5 * (e + 1.0 / e), 1.0 + 1e-7)
    rt = _arccosh(time)
    nt = jnp.sqrt(jnp.sum(ys * ys, axis=-1, keepdims=True))
    vt_ref[...] = rt * ys / jnp.maximum(nt, 1e-7)


def _tail_body(s_ref, vt_ref, c_ref, out_ref):
    vt = vt_ref[...]
    s = s_ref[pl.ds(0, N), :] + s_ref[pl.ds(N, N), :] + vt
    c = jnp.sum(c_ref[...], axis=0)
    cnt = c.reshape(_NPAD)[:N][:, None] + 1.0
    mean = s / jnp.maximum(cnt, 1.0)
    xa, _ = _exp0_spatial(mean)
    v = jnp.maximum(_tangent_of(xa), 0.0)
    xo, _ = _exp0_spatial(v)
    x0 = jnp.sqrt(1.0 + jnp.sum(xo * xo, axis=-1, keepdims=True))
    out_ref[...] = jnp.concatenate([x0, xo], axis=-1)


def _sc_aggregate(vt, src, dst, zrow, zflat):
    mesh = plsc.VectorSubcoreMesh(core_axis_name="c", subcore_axis_name="s")

    @functools.partial(
        pl.kernel,
        out_type=(jax.ShapeDtypeStruct((_NC * N, D), jnp.float32),
                  jax.ShapeDtypeStruct((_NW * _NPAD,), jnp.float32)),
        mesh=mesh,
        compiler_params=pltpu.CompilerParams(needs_layout_passes=False),
        scratch_types=[
            [pltpu.VMEM((_C,), jnp.int32)] * _NBUF,
            [pltpu.VMEM((_C, D), jnp.float32)] * _NBUF,
            pltpu.VMEM((_EPT,), jnp.int32),
            pltpu.VMEM((_NPAD,), jnp.float32),
            pltpu.VMEM_SHARED((N, D), jnp.float32),
            pltpu.SemaphoreType.DMA,
            pltpu.SemaphoreType.DMA,
            pltpu.SemaphoreType.DMA,
        ],
    )
    def agg(vt_hbm, src_hbm, dst_hbm, zrow_hbm, zflat_hbm,
            sums_hbm, cnts_hbm,
            src_v, rows_v, dall_v, cnt_v, acc_sh,
            sem_i, sem_g, sem_s):
        cid = lax.axis_index("c")
        sid = lax.axis_index("s")
        wid = cid * _NS + sid

        pltpu.sync_copy(zflat_hbm, cnt_v)
        pltpu.sync_copy(dst_hbm.at[pl.ds(wid * _EPT, _EPT)], dall_v)

        pltpu.sync_copy(zrow_hbm, rows_v[0])
        for k in range(_KMAX):
            u = sid + _NS * k

            @pl.when(u < _NU)
            def _zero_unit():
                pltpu.sync_copy(rows_v[0], acc_sh.at[pl.ds(u * _U, _U)])

        plsc.subcore_barrier()

        def group(g, carry):
            base0 = wid * _EPT + g * (_C * _NBUF)

            @pl.when(g > 0)
            def _drain_prev():
                for b in range(_NBUF):
                    pltpu.make_async_copy(
                        rows_v[b], acc_sh.at[dall_v.at[pl.ds(0, _C)]],
                        sem_s).wait()

            hi, hg = [], []
            for b in range(_NBUF):
                hi.append(pltpu.async_copy(
                    src_hbm.at[pl.ds(base0 + b * _C, _C)], src_v[b], sem_i))
            for b in range(_NBUF):
                hi[b].wait()
                hg.append(pltpu.async_copy(
                    vt_hbm.at[src_v[b]], rows_v[b], sem_g))
            for b in range(_NBUF):
                hg[b].wait()
                off = (g * _NBUF + b) * _C
                pltpu.async_copy(
                    rows_v[b], acc_sh.at[dall_v.at[pl.ds(off, _C)]],
                    sem_s, add=True)
            return carry

        lax.fori_loop(0, _NGRP, group, 0)
        for b in range(_NBUF):
            pltpu.make_async_copy(
                rows_v[b], acc_sh.at[dall_v.at[pl.ds(0, _C)]], sem_s).wait()

        one16 = jnp.ones((_L,), jnp.float32)

        def cloop(k, carry):
            dvals = dall_v[pl.ds(k * _L, _L)]
            plsc.addupdate_scatter(cnt_v, [dvals], one16)
            return carry

        lax.fori_loop(0, _EPT // _L, cloop, 0)
        plsc.subcore_barrier()

        for k in range(_KMAX):
            u = sid + _NS * k

            @pl.when(u < _NU)
            def _write_unit():
                pltpu.sync_copy(acc_sh.at[pl.ds(u * _U, _U)], rows_v[0])
                pltpu.sync_copy(rows_v[0], sums_hbm.at[pl.ds(cid * N + u * _U, _U)])

        pltpu.sync_copy(cnt_v, cnts_hbm.at[pl.ds(wid * _NPAD, _NPAD)])

    return agg(vt, src, dst, zrow, zflat)


def kernel(x, edge_index, W, b):
    xs = x[:, 1:]
    b2 = b.reshape(1, D)
    bn = 1000
    grid = N // bn

    vt = pl.pallas_call(
        _transform_body,
        grid=(grid,),
        in_specs=[
            pl.BlockSpec((bn, D), lambda i: (i, 0)),
            pl.BlockSpec((D, D), lambda i: (0, 0)),
            pl.BlockSpec((1, D), lambda i: (0, 0)),
        ],
        out_specs=pl.BlockSpec((bn, D), lambda i: (i, 0)),
        out_shape=jax.ShapeDtypeStruct((N, D), jnp.float32),
    )(xs, W, b2)

    zrow = jnp.zeros((_U, D), jnp.float32)
    zflat = jnp.zeros((_NPAD,), jnp.float32)
    sums, cnts = _sc_aggregate(vt, edge_index[0], edge_index[1], zrow, zflat)
    cnts = cnts.reshape(_NW, _NPAD // D, D)

    out = pl.pallas_call(
        _tail_body,
        out_shape=jax.ShapeDtypeStruct((N, D + 1), jnp.float32),
    )(sums, vt, cnts)

    return out


# speedup vs baseline: 13.4879x; 1.0308x over previous
import functools

import jax
import jax.numpy as jnp
from jax import lax
from jax.experimental import pallas as pl
from jax.experimental.pallas import tpu as pltpu
from jax.experimental.pallas import tpu_sc as plsc

N = 10000
E = 320000
D = 128

_NC = 2
_NS = 16
_NW = _NC * _NS
_EPT = E // _NW
_C = 40
_NBUF = 5
_NGRP = _EPT // (_C * _NBUF)
_U = 40
_NU = N // _U
_KMAX = -(-_NU // _NS)
_L = 16
_NPAD = 10240


def _arccosh(x):
    return jnp.log(x + jnp.sqrt((x - 1.0) * (x + 1.0)))


def _tangent_of(y_s):
    nsq = jnp.sum(y_s * y_s, axis=-1, keepdims=True)
    x0 = jnp.maximum(jnp.sqrt(1.0 + nsq), 1.0 + 1e-7)
    r = _arccosh(x0)
    n = jnp.sqrt(nsq)
    return r * y_s / jnp.maximum(n, 1e-7)


def _exp0_spatial(v):
    rsq = jnp.sum(v * v, axis=-1, keepdims=True)
    r = jnp.sqrt(rsq)
    safe_r = jnp.maximum(r, 1e-7)
    e = jnp.exp(r)
    sinh = 0.5 * (e - 1.0 / e)
    coef = jnp.where(r > 1e-7, sinh / safe_r, 1.0)
    return coef * v, r


def _transform_body(x_ref, w_ref, b_ref, vt_ref):
    xs = x_ref[:, 1:]
    u = _tangent_of(xs)
    vt_ref[...] = lax.dot_general(u, w_ref[...], (((1,), (1,)), ((), ())),
                                  preferred_element_type=jnp.float32) + b_ref[...]


def _tail_body(s_ref, vt_ref, c_ref, out_ref):
    vt = vt_ref[...]
    s = s_ref[pl.ds(0, N), :] + s_ref[pl.ds(N, N), :] + vt
    c = jnp.sum(c_ref[...], axis=0)
    cnt = c.reshape(_NPAD)[:N][:, None] + 1.0
    mean = s / jnp.maximum(cnt, 1.0)
    v = jnp.maximum(mean, 0.0)
    xo, _ = _exp0_spatial(v)
    x0 = jnp.sqrt(1.0 + jnp.sum(xo * xo, axis=-1, keepdims=True))
    out_ref[...] = jnp.concatenate([x0, xo], axis=-1)


def _sc_aggregate(vt, src, dst, z80):
    mesh = plsc.VectorSubcoreMesh(core_axis_name="c", subcore_axis_name="s")

    @functools.partial(
        pl.kernel,
        out_type=(jax.ShapeDtypeStruct((_NC * N, D), jnp.float32),
                  jax.ShapeDtypeStruct((_NW, _NPAD // D, D), jnp.float32)),
        mesh=mesh,
        compiler_params=pltpu.CompilerParams(needs_layout_passes=False),
        scratch_types=[
            [pltpu.VMEM((_C,), jnp.int32)] * _NBUF,
            [pltpu.VMEM((_C, D), jnp.float32)] * _NBUF,
            pltpu.VMEM((_EPT,), jnp.int32),
            pltpu.VMEM((_NPAD // D, D), jnp.float32),
            pltpu.VMEM_SHARED((N, D), jnp.float32),
            pltpu.SemaphoreType.DMA,
            pltpu.SemaphoreType.DMA,
            pltpu.SemaphoreType.DMA,
        ],
    )
    def agg(vt_hbm, src_hbm, dst_hbm, z80_hbm,
            sums_hbm, cnts_hbm,
            src_v, rows_v, dall_v, cnt_v, acc_sh,
            sem_i, sem_g, sem_s):
        cid = lax.axis_index("c")
        sid = lax.axis_index("s")
        wid = cid * _NS + sid

        pltpu.sync_copy(z80_hbm, cnt_v)
        pltpu.sync_copy(dst_hbm.at[pl.ds(wid * _EPT, _EPT)], dall_v)

        pltpu.sync_copy(z80_hbm.at[pl.ds(0, _U)], rows_v[0])
        for k in range(_KMAX):
            u = sid + _NS * k

            @pl.when(u < _NU)
            def _zero_unit():
                pltpu.sync_copy(rows_v[0], acc_sh.at[pl.ds(u * _U, _U)])

        plsc.subcore_barrier()

        def group(g, carry):
            base0 = wid * _EPT + g * (_C * _NBUF)

            @pl.when(g > 0)
            def _drain_prev():
                for b in range(_NBUF):
                    pltpu.make_async_copy(
                        rows_v[b], acc_sh.at[dall_v.at[pl.ds(0, _C)]],
                        sem_s).wait()

            hi, hg = [], []
            for b in range(_NBUF):
                hi.append(pltpu.async_copy(
                    src_hbm.at[pl.ds(base0 + b * _C, _C)], src_v[b], sem_i))
            for b in range(_NBUF):
                hi[b].wait()
                hg.append(pltpu.async_copy(
                    vt_hbm.at[src_v[b]], rows_v[b], sem_g))
            for b in range(_NBUF):
                hg[b].wait()
                off = (g * _NBUF + b) * _C
                pltpu.async_copy(
                    rows_v[b], acc_sh.at[dall_v.at[pl.ds(off, _C)]],
                    sem_s, add=True)
            return carry

        lax.fori_loop(0, _NGRP, group, 0)
        for b in range(_NBUF):
            pltpu.make_async_copy(
                rows_v[b], acc_sh.at[dall_v.at[pl.ds(0, _C)]], sem_s).wait()

        one16 = jnp.ones((_L,), jnp.float32)

        def cloop(k, carry):
            dvals = dall_v[pl.ds(k * _L, _L)]
            plsc.addupdate_scatter(
                cnt_v, [lax.shift_right_logical(dvals, 7),
                        lax.bitwise_and(dvals, 127)], one16)
            return carry

        lax.fori_loop(0, _EPT // _L, cloop, 0)
        plsc.subcore_barrier()

        for k in range(_KMAX):
            u = sid + _NS * k

            @pl.when(u < _NU)
            def _write_unit():
                pltpu.sync_copy(acc_sh.at[pl.ds(u * _U, _U)], rows_v[0])
                pltpu.sync_copy(rows_v[0], sums_hbm.at[pl.ds(cid * N + u * _U, _U)])

        pltpu.sync_copy(cnt_v, cnts_hbm.at[wid])

    return agg(vt, src, dst, z80)


def kernel(x, edge_index, W, b):
    b2 = b.reshape(1, D)
    bn = 1000
    grid = N // bn

    vt = pl.pallas_call(
        _transform_body,
        grid=(grid,),
        in_specs=[
            pl.BlockSpec((bn, D + 1), lambda i: (i, 0)),
            pl.BlockSpec((D, D), lambda i: (0, 0)),
            pl.BlockSpec((1, D), lambda i: (0, 0)),
        ],
        out_specs=pl.BlockSpec((bn, D), lambda i: (i, 0)),
        out_shape=jax.ShapeDtypeStruct((N, D), jnp.float32),
    )(x, W, b2)

    z80 = jnp.zeros((_NPAD // D, D), jnp.float32)
    sums, cnts = _sc_aggregate(vt, edge_index[0], edge_index[1], z80)

    out = pl.pallas_call(
        _tail_body,
        out_shape=jax.ShapeDtypeStruct((N, D + 1), jnp.float32),
    )(sums, vt, cnts)

    return out


# speedup vs baseline: 14.7988x; 1.0972x over previous
import functools

import jax
import jax.numpy as jnp
from jax import lax
from jax.experimental import pallas as pl
from jax.experimental.pallas import tpu as pltpu
from jax.experimental.pallas import tpu_sc as plsc

N = 10000
E = 320000
D = 128

_NC = 2
_NS = 16
_NW = _NC * _NS
_EPT = E // _NW
_C = 40
_NBUF = 5
_NGRP = _EPT // (_C * _NBUF)
_U = 40
_NU = N // _U
_KMAX = -(-_NU // _NS)
_L = 16
_NPAD = 10240


def _arccosh(x):
    return jnp.log(x + jnp.sqrt((x - 1.0) * (x + 1.0)))


def _tangent_of(y_s):
    nsq = jnp.sum(y_s * y_s, axis=-1, keepdims=True)
    x0 = jnp.maximum(jnp.sqrt(1.0 + nsq), 1.0 + 1e-7)
    r = _arccosh(x0)
    n = jnp.sqrt(nsq)
    return r * y_s / jnp.maximum(n, 1e-7)


def _exp0_spatial(v):
    rsq = jnp.sum(v * v, axis=-1, keepdims=True)
    r = jnp.sqrt(rsq)
    safe_r = jnp.maximum(r, 1e-7)
    e = jnp.exp(r)
    sinh = 0.5 * (e - 1.0 / e)
    coef = jnp.where(r > 1e-7, sinh / safe_r, 1.0)
    return coef * v, r


def _transform_body(x_ref, w_ref, b_ref, vt_ref):
    xs = x_ref[:, 1:]
    u = _tangent_of(xs)
    vt_ref[...] = lax.dot_general(u, w_ref[...], (((1,), (1,)), ((), ())),
                                  preferred_element_type=jnp.float32) + b_ref[...]


def _tail_body(s_ref, vt_ref, c_ref, out_ref):
    vt = vt_ref[...]
    s = s_ref[pl.ds(0, N), :] + s_ref[pl.ds(N, N), :] + vt
    c = jnp.sum(c_ref[...], axis=0)
    cnt = c.reshape(_NPAD)[:N][:, None] + 1.0
    mean = s / jnp.maximum(cnt, 1.0)
    v = jnp.maximum(mean, 0.0)
    xo, _ = _exp0_spatial(v)
    x0 = jnp.sqrt(1.0 + jnp.sum(xo * xo, axis=-1, keepdims=True))
    out_ref[...] = jnp.concatenate([x0, xo], axis=-1)


def _sc_aggregate(vt, src, dst, z80):
    mesh = plsc.VectorSubcoreMesh(core_axis_name="c", subcore_axis_name="s")

    @functools.partial(
        pl.kernel,
        out_type=(jax.ShapeDtypeStruct((_NC * N, D), jnp.float32),
                  jax.ShapeDtypeStruct((_NW, _NPAD // D, D), jnp.float32)),
        mesh=mesh,
        compiler_params=pltpu.CompilerParams(needs_layout_passes=False),
        scratch_types=[
            [pltpu.VMEM((_C,), jnp.int32)] * _NBUF,
            [pltpu.VMEM((_C, D), jnp.float32)] * _NBUF,
            pltpu.VMEM((_EPT,), jnp.int32),
            pltpu.VMEM((_NPAD // D, D), jnp.float32),
            pltpu.VMEM_SHARED((N, D), jnp.float32),
            pltpu.SemaphoreType.DMA,
            pltpu.SemaphoreType.DMA,
            pltpu.SemaphoreType.DMA,
        ],
    )
    def agg(vt_hbm, src_hbm, dst_hbm, z80_hbm,
            sums_hbm, cnts_hbm,
            src_v, rows_v, dall_v, cnt_v, acc_sh,
            sem_i, sem_g, sem_s):
        cid = lax.axis_index("c")
        sid = lax.axis_index("s")
        wid = cid * _NS + sid

        pltpu.sync_copy(z80_hbm, cnt_v)
        pltpu.sync_copy(dst_hbm.at[pl.ds(wid * _EPT, _EPT)], dall_v)

        pltpu.sync_copy(z80_hbm.at[pl.ds(0, _U)], rows_v[0])
        for k in range(_KMAX):
            u = sid + _NS * k

            @pl.when(u < _NU)
            def _zero_unit():
                pltpu.sync_copy(rows_v[0], acc_sh.at[pl.ds(u * _U, _U)])

        plsc.subcore_barrier()

        for b in range(_NBUF):
            pltpu.async_copy(
                src_hbm.at[pl.ds(wid * _EPT + b * _C, _C)], src_v[b], sem_i)

        def group(g, carry):
            base0 = wid * _EPT + g * (_C * _NBUF)

            @pl.when(g > 0)
            def _drain_prev():
                for b in range(_NBUF):
                    pltpu.make_async_copy(
                        rows_v[b], acc_sh.at[dall_v.at[pl.ds(0, _C)]],
                        sem_s).wait()

            hg = []
            for b in range(_NBUF):
                pltpu.make_async_copy(
                    src_hbm.at[pl.ds(base0 + b * _C, _C)], src_v[b],
                    sem_i).wait()
                hg.append(pltpu.async_copy(
                    vt_hbm.at[src_v[b]], rows_v[b], sem_g))
            for b in range(_NBUF):
                hg[b].wait()
                off = (g * _NBUF + b) * _C
                pltpu.async_copy(
                    rows_v[b], acc_sh.at[dall_v.at[pl.ds(off, _C)]],
                    sem_s, add=True)

                @pl.when(g < _NGRP - 1)
                def _prefetch_idx(b=b):
                    pltpu.async_copy(
                        src_hbm.at[pl.ds(base0 + (_NBUF + b) * _C, _C)],
                        src_v[b], sem_i)

            return carry

        lax.fori_loop(0, _NGRP, group, 0)
        for b in range(_NBUF):
            pltpu.make_async_copy(
                rows_v[b], acc_sh.at[dall_v.at[pl.ds(0, _C)]], sem_s).wait()

        one16 = jnp.ones((_L,), jnp.float32)

        def cloop(k, carry):
            dvals = dall_v[pl.ds(k * _L, _L)]
            plsc.addupdate_scatter(
                cnt_v, [lax.shift_right_logical(dvals, 7),
                        lax.bitwise_and(dvals, 127)], one16)
            return carry

        lax.fori_loop(0, _EPT // _L, cloop, 0)
        plsc.subcore_barrier()

        for k in range(_KMAX):
            u = sid + _NS * k

            @pl.when(u < _NU)
            def _write_unit():
                pltpu.sync_copy(acc_sh.at[pl.ds(u * _U, _U)], rows_v[0])
                pltpu.sync_copy(rows_v[0], sums_hbm.at[pl.ds(cid * N + u * _U, _U)])

        pltpu.sync_copy(cnt_v, cnts_hbm.at[wid])

    return agg(vt, src, dst, z80)


def kernel(x, edge_index, W, b):
    b2 = b.reshape(1, D)
    bn = 1000
    grid = N // bn

    vt = pl.pallas_call(
        _transform_body,
        grid=(grid,),
        in_specs=[
            pl.BlockSpec((bn, D + 1), lambda i: (i, 0)),
            pl.BlockSpec((D, D), lambda i: (0, 0)),
            pl.BlockSpec((1, D), lambda i: (0, 0)),
        ],
        out_specs=pl.BlockSpec((bn, D), lambda i: (i, 0)),
        out_shape=jax.ShapeDtypeStruct((N, D), jnp.float32),
    )(x, W, b2)

    z80 = jnp.zeros((_NPAD // D, D), jnp.float32)
    sums, cnts = _sc_aggregate(vt, edge_index[0], edge_index[1], z80)

    out = pl.pallas_call(
        _tail_body,
        out_shape=jax.ShapeDtypeStruct((N, D + 1), jnp.float32),
    )(sums, vt, cnts)

    return out


# speedup vs baseline: 15.4669x; 1.0451x over previous
import functools

import jax
import jax.numpy as jnp
from jax import lax
from jax.experimental import pallas as pl
from jax.experimental.pallas import tpu as pltpu
from jax.experimental.pallas import tpu_sc as plsc

N = 10000
E = 320000
D = 128

_NC = 2
_NS = 16
_NW = _NC * _NS
_EPT = E // _NW
_C = 40
_NBUF = 5
_NGRP = _EPT // (_C * _NBUF)
_U = 40
_NU = N // _U
_KMAX = -(-_NU // _NS)
_L = 16
_NPAD = 10240


def _arccosh(x):
    return jnp.log(x + jnp.sqrt((x - 1.0) * (x + 1.0)))


def _tangent_of(y_s):
    nsq = jnp.sum(y_s * y_s, axis=-1, keepdims=True)
    x0 = jnp.maximum(jnp.sqrt(1.0 + nsq), 1.0 + 1e-7)
    r = _arccosh(x0)
    n = jnp.sqrt(nsq)
    return r * y_s / jnp.maximum(n, 1e-7)


def _exp0_spatial(v):
    rsq = jnp.sum(v * v, axis=-1, keepdims=True)
    r = jnp.sqrt(rsq)
    safe_r = jnp.maximum(r, 1e-7)
    e = jnp.exp(r)
    sinh = 0.5 * (e - 1.0 / e)
    coef = jnp.where(r > 1e-7, sinh / safe_r, 1.0)
    return coef * v, r


def _transform_body(x_ref, w_ref, b_ref, vt_ref):
    xs = x_ref[:, 1:]
    u = _tangent_of(xs)
    vt_ref[...] = lax.dot_general(u, w_ref[...], (((1,), (1,)), ((), ())),
                                  preferred_element_type=jnp.float32) + b_ref[...]


def _tail_body(s_ref, vt_ref, c_ref, out_ref):
    vt = vt_ref[...]
    s = s_ref[pl.ds(0, N), :] + s_ref[pl.ds(N, N), :] + vt
    c = jnp.sum(c_ref[...], axis=0)
    cnt = c.reshape(_NPAD)[:N][:, None] + 1.0
    mean = s / jnp.maximum(cnt, 1.0)
    v = jnp.maximum(mean, 0.0)
    xo, _ = _exp0_spatial(v)
    x0 = jnp.sqrt(1.0 + jnp.sum(xo * xo, axis=-1, keepdims=True))
    out_ref[...] = jnp.concatenate([x0, xo], axis=-1)


def _sc_aggregate(vt, src, dst, z80):
    mesh = plsc.VectorSubcoreMesh(core_axis_name="c", subcore_axis_name="s")

    @functools.partial(
        pl.kernel,
        out_type=(jax.ShapeDtypeStruct((_NC * N, D), jnp.float32),
                  jax.ShapeDtypeStruct((_NW, _NPAD // D, D), jnp.float32)),
        mesh=mesh,
        compiler_params=pltpu.CompilerParams(needs_layout_passes=False),
        scratch_types=[
            [pltpu.VMEM((_C,), jnp.int32)] * _NBUF,
            [pltpu.VMEM((_C, D), jnp.float32)] * _NBUF,
            pltpu.VMEM((_EPT,), jnp.int32),
            pltpu.VMEM((_NPAD // D, D), jnp.float32),
            pltpu.VMEM_SHARED((N, D), jnp.float32),
            pltpu.SemaphoreType.DMA,
            pltpu.SemaphoreType.DMA,
            pltpu.SemaphoreType.DMA,
        ],
    )
    def agg(vt_hbm, src_hbm, dst_hbm, z80_hbm,
            sums_hbm, cnts_hbm,
            src_v, rows_v, dall_v, cnt_v, acc_sh,
            sem_i, sem_g, sem_s):
        cid = lax.axis_index("c")
        sid = lax.axis_index("s")
        wid = cid * _NS + sid

        pltpu.sync_copy(z80_hbm, cnt_v)
        pltpu.sync_copy(dst_hbm.at[pl.ds(wid * _EPT, _EPT)], dall_v)

        pltpu.sync_copy(z80_hbm.at[pl.ds(0, _U)], rows_v[0])
        for k in range(_KMAX):
            u = sid + _NS * k

            @pl.when(u < _NU)
            def _zero_unit():
                pltpu.sync_copy(rows_v[0], acc_sh.at[pl.ds(u * _U, _U)])

        plsc.subcore_barrier()

        for b in range(_NBUF):
            pltpu.async_copy(
                src_hbm.at[pl.ds(wid * _EPT + b * _C, _C)], src_v[b], sem_i)

        def group(g, carry):
            base0 = wid * _EPT + g * (_C * _NBUF)

            hg = []
            for b in range(_NBUF):
                @pl.when(g > 0)
                def _drain_prev(b=b):
                    pltpu.make_async_copy(
                        rows_v[b], acc_sh.at[dall_v.at[pl.ds(0, _C)]],
                        sem_s).wait()

                pltpu.make_async_copy(
                    src_hbm.at[pl.ds(base0 + b * _C, _C)], src_v[b],
                    sem_i).wait()
                hg.append(pltpu.async_copy(
                    vt_hbm.at[src_v[b]], rows_v[b], sem_g))
            for b in range(_NBUF):
                hg[b].wait()
                off = (g * _NBUF + b) * _C
                pltpu.async_copy(
                    rows_v[b], acc_sh.at[dall_v.at[pl.ds(off, _C)]],
                    sem_s, add=True)

                @pl.when(g < _NGRP - 1)
                def _prefetch_idx(b=b):
                    pltpu.async_copy(
                        src_hbm.at[pl.ds(base0 + (_NBUF + b) * _C, _C)],
                        src_v[b], sem_i)

            return carry

        lax.fori_loop(0, _NGRP, group, 0)
        for b in range(_NBUF):
            pltpu.make_async_copy(
                rows_v[b], acc_sh.at[dall_v.at[pl.ds(0, _C)]], sem_s).wait()

        one16 = jnp.ones((_L,), jnp.float32)

        def cloop(k, carry):
            dvals = dall_v[pl.ds(k * _L, _L)]
            plsc.addupdate_scatter(
                cnt_v, [lax.shift_right_logical(dvals, 7),
                        lax.bitwise_and(dvals, 127)], one16)
            return carry

        lax.fori_loop(0, _EPT // _L, cloop, 0)
        plsc.subcore_barrier()

        for k in range(_KMAX):
            u = sid + _NS * k

            @pl.when(u < _NU)
            def _write_unit():
                pltpu.sync_copy(acc_sh.at[pl.ds(u * _U, _U)], rows_v[0])
                pltpu.sync_copy(rows_v[0], sums_hbm.at[pl.ds(cid * N + u * _U, _U)])

        pltpu.sync_copy(cnt_v, cnts_hbm.at[wid])

    return agg(vt, src, dst, z80)


def kernel(x, edge_index, W, b):
    b2 = b.reshape(1, D)
    bn = 1000
    grid = N // bn

    vt = pl.pallas_call(
        _transform_body,
        grid=(grid,),
        in_specs=[
            pl.BlockSpec((bn, D + 1), lambda i: (i, 0)),
            pl.BlockSpec((D, D), lambda i: (0, 0)),
            pl.BlockSpec((1, D), lambda i: (0, 0)),
        ],
        out_specs=pl.BlockSpec((bn, D), lambda i: (i, 0)),
        out_shape=jax.ShapeDtypeStruct((N, D), jnp.float32),
    )(x, W, b2)

    z80 = jnp.zeros((_NPAD // D, D), jnp.float32)
    sums, cnts = _sc_aggregate(vt, edge_index[0], edge_index[1], z80)

    out = pl.pallas_call(
        _tail_body,
        out_shape=jax.ShapeDtypeStruct((N, D + 1), jnp.float32),
    )(sums, vt, cnts)

    return out
